# Initial kernel scaffold; baseline (speedup 1.0000x reference)
import jax
import jax.numpy as jnp
from jax.experimental import pallas as pl


def kernel(x, in_f, fc1_w, fc1_b, fc2_w, fc2_b, phi_w, psi_w, alpha_w, dpt1_w, dpt1_b, dpt2_w, dpt2_b, gam1_w, gam1_b, gam2_w, gam2_b, del1_w, del1_b, del2_w, del2_b):
    raise NotImplementedError("write your pallas kernel here")


# speedup vs baseline: 11.2838x; 11.2838x over previous
import functools

import jax
import jax.numpy as jnp
from jax import lax
from jax.experimental import pallas as pl
from jax.experimental.pallas import tpu as pltpu
from jax.experimental.pallas import tpu_sc as plsc

BATCH = 2
NPTS = 2048
KNN = 16
DIM = 256
PDIM = 64
XPAD = 16

ROWS_A = 256
ROWS_C = 64

SC_CORES = 2
SC_SUBCORES = 16
SC_WORKERS = SC_CORES * SC_SUBCORES
TOTAL_PAIRS = BATCH * NPTS * KNN
PAIRS_PER_WORKER = TOTAL_PAIRS // SC_WORKERS
GATHER_CHUNK = 128
N_CHUNKS = PAIRS_PER_WORKER // GATHER_CHUNK


def _stage1_body(x_ref, xt_ref, inf_ref,
                 fc1wT, fc1b, phiwT, psiwT, alphawT,
                 dpt1wT, dpt1b, dpt2wT, dpt2b, del1pT,
                 idx_out, pd_out, s_out, a_out, e_out):
    b = pl.program_id(0)
    xb = x_ref[0]
    xall_t = xt_ref[0]
    d = -2.0 * jnp.dot(xb, xall_t, preferred_element_type=jnp.float32)
    d = d + jnp.sum(xb * xb, axis=1, keepdims=True)
    d = d + jnp.sum(xall_t * xall_t, axis=0, keepdims=True)

    iota = lax.broadcasted_iota(jnp.int32, d.shape, 1)
    cols = []
    for _ in range(KNN):
        m = jnp.min(d, axis=1, keepdims=True)
        ii = jnp.min(jnp.where(d == m, iota, jnp.int32(1 << 30)), axis=1)
        cols.append(ii)
        d = jnp.where(iota == ii[:, None], jnp.float32(jnp.inf), d)
    idx = jnp.stack(cols, axis=1)
    idx_out[0] = idx + b * NPTS

    e_out[0] = jnp.dot(xb, del1pT[...], preferred_element_type=jnp.float32)
    f = jnp.dot(inf_ref[0], fc1wT[...], preferred_element_type=jnp.float32) + fc1b[...]
    dr = jnp.maximum(jnp.dot(f, dpt1wT[...], preferred_element_type=jnp.float32) + dpt1b[...], 0.0)
    dr = jnp.dot(dr, dpt2wT[...], preferred_element_type=jnp.float32) + dpt2b[...]
    pd_out[0] = jnp.dot(f, phiwT[...], preferred_element_type=jnp.float32) + dr
    s_out[0] = jnp.dot(f, psiwT[...], preferred_element_type=jnp.float32)
    a_out[0] = jnp.dot(f, alphawT[...], preferred_element_type=jnp.float32)


def _sc_gather_body(idx_hbm, s_hbm, a_hbm, e_hbm,
                    sg_out, ag_out, eg_out,
                    idx_v, sbuf, abuf, ebuf, sem_s, sem_a, sem_e):
    wid = lax.axis_index("s") * SC_CORES + lax.axis_index("c")
    base = wid * PAIRS_PER_WORKER
    pltpu.sync_copy(idx_hbm.at[pl.ds(base, PAIRS_PER_WORKER)], idx_v)

    def chunk(c, carry):
        off = c * GATHER_CHUNK
        isl = idx_v.at[pl.ds(off, GATHER_CHUNK)]
        cp_s = pltpu.async_copy(s_hbm.at[isl], sbuf, sem_s)
        cp_a = pltpu.async_copy(a_hbm.at[isl], abuf, sem_a)
        cp_e = pltpu.async_copy(e_hbm.at[isl], ebuf, sem_e)
        cp_s.wait()
        pltpu.sync_copy(sbuf, sg_out.at[pl.ds(base + off, GATHER_CHUNK)])
        cp_a.wait()
        pltpu.sync_copy(abuf, ag_out.at[pl.ds(base + off, GATHER_CHUNK)])
        cp_e.wait()
        pltpu.sync_copy(ebuf, eg_out.at[pl.ds(base + off, GATHER_CHUNK)])
        return carry

    lax.fori_loop(0, N_CHUNKS, chunk, 0)


def _stage3_body(e_ref, inf_ref, pd_ref, sg_ref, ag_ref, eg_ref,
                 del1b, del2wT, del2b,
                 gam1wT, gam1b, gam2wT, gam2b,
                 fc2wT, fc2b, out_ref):
    nb = ROWS_C
    en = e_ref[0]
    eg = eg_ref[0].reshape(nb, KNN, DIM)
    h = (en[:, None, :] - eg).reshape(nb * KNN, DIM) + del1b[...]
    u = jnp.maximum(h, 0.0)
    delta = jnp.dot(u, del2wT[...], preferred_element_type=jnp.float32) + del2b[...]

    pd = pd_ref[0]
    t = (pd[:, None, :] - sg_ref[0].reshape(nb, KNN, DIM)).reshape(nb * KNN, DIM) + delta
    g1 = jnp.maximum(jnp.dot(t, gam1wT[...], preferred_element_type=jnp.float32) + gam1b[...], 0.0)
    gamma = jnp.dot(g1, gam2wT[...], preferred_element_type=jnp.float32) + gam2b[...]

    s = gamma.reshape(nb, KNN, DIM) * (1.0 / (KNN ** 0.5))
    smax = jnp.max(s, axis=1, keepdims=True)
    e = jnp.exp(s - smax)
    rho = e / jnp.sum(e, axis=1, keepdims=True)

    ad = ag_ref[0].reshape(nb, KNN, DIM) + delta.reshape(nb, KNN, DIM)
    y = jnp.sum(rho * ad, axis=1)
    out_ref[0] = jnp.dot(y, fc2wT[...], preferred_element_type=jnp.float32) + fc2b[...] + inf_ref[0]


def _full(shape):
    nd = len(shape)
    return pl.BlockSpec(shape, lambda b, i: (0,) * nd)


def kernel(x, in_f, fc1_w, fc1_b, fc2_w, fc2_b, phi_w, psi_w, alpha_w,
           dpt1_w, dpt1_b, dpt2_w, dpt2_b, gam1_w, gam1_b, gam2_w, gam2_b,
           del1_w, del1_b, del2_w, del2_b):
    f32 = jnp.float32
    xpad = jnp.pad(x, ((0, 0), (0, 0), (0, XPAD - 3)))
    xt = jnp.transpose(xpad, (0, 2, 1))
    del1p = jnp.pad(del1_w, ((0, 0), (0, XPAD - 3)))

    row = lambda v: v.reshape(1, -1)

    grid_a = (BATCH, NPTS // ROWS_A)
    idx, pdt, st, at, et = pl.pallas_call(
        _stage1_body,
        grid=grid_a,
        in_specs=[
            pl.BlockSpec((1, ROWS_A, XPAD), lambda b, i: (b, i, 0)),
            pl.BlockSpec((1, XPAD, NPTS), lambda b, i: (b, 0, 0)),
            pl.BlockSpec((1, ROWS_A, PDIM), lambda b, i: (b, i, 0)),
            _full((PDIM, DIM)), _full((1, DIM)),
            _full((DIM, DIM)), _full((DIM, DIM)), _full((DIM, DIM)),
            _full((DIM, DIM)), _full((1, DIM)),
            _full((DIM, DIM)), _full((1, DIM)),
            _full((XPAD, DIM)),
        ],
        out_specs=[
            pl.BlockSpec((1, ROWS_A, KNN), lambda b, i: (b, i, 0)),
            pl.BlockSpec((1, ROWS_A, DIM), lambda b, i: (b, i, 0)),
            pl.BlockSpec((1, ROWS_A, DIM), lambda b, i: (b, i, 0)),
            pl.BlockSpec((1, ROWS_A, DIM), lambda b, i: (b, i, 0)),
            pl.BlockSpec((1, ROWS_A, DIM), lambda b, i: (b, i, 0)),
        ],
        out_shape=[
            jax.ShapeDtypeStruct((BATCH, NPTS, KNN), jnp.int32),
            jax.ShapeDtypeStruct((BATCH, NPTS, DIM), f32),
            jax.ShapeDtypeStruct((BATCH, NPTS, DIM), f32),
            jax.ShapeDtypeStruct((BATCH, NPTS, DIM), f32),
            jax.ShapeDtypeStruct((BATCH, NPTS, DIM), f32),
        ],
    )(xpad, xt, in_f,
      fc1_w.T, row(fc1_b), phi_w.T, psi_w.T, alpha_w.T,
      dpt1_w.T, row(dpt1_b), dpt2_w.T, row(dpt2_b), del1p.T)

    sc_gather = functools.partial(
        pl.kernel,
        out_type=[
            jax.ShapeDtypeStruct((TOTAL_PAIRS, DIM), f32),
            jax.ShapeDtypeStruct((TOTAL_PAIRS, DIM), f32),
            jax.ShapeDtypeStruct((TOTAL_PAIRS, DIM), f32),
        ],
        mesh=plsc.VectorSubcoreMesh(core_axis_name="c", subcore_axis_name="s"),
        scratch_types=[
            pltpu.VMEM((PAIRS_PER_WORKER,), jnp.int32),
            pltpu.VMEM((GATHER_CHUNK, DIM), f32),
            pltpu.VMEM((GATHER_CHUNK, DIM), f32),
            pltpu.VMEM((GATHER_CHUNK, DIM), f32),
            pltpu.SemaphoreType.DMA,
            pltpu.SemaphoreType.DMA,
            pltpu.SemaphoreType.DMA,
        ],
    )(_sc_gather_body)
    sg, ag, eg = sc_gather(
        idx.reshape(TOTAL_PAIRS),
        st.reshape(BATCH * NPTS, DIM),
        at.reshape(BATCH * NPTS, DIM),
        et.reshape(BATCH * NPTS, DIM),
    )

    grid_c = (BATCH, NPTS // ROWS_C)
    out = pl.pallas_call(
        _stage3_body,
        grid=grid_c,
        in_specs=[
            pl.BlockSpec((1, ROWS_C, DIM), lambda b, i: (b, i, 0)),
            pl.BlockSpec((1, ROWS_C, PDIM), lambda b, i: (b, i, 0)),
            pl.BlockSpec((1, ROWS_C, DIM), lambda b, i: (b, i, 0)),
            pl.BlockSpec((1, ROWS_C * KNN, DIM), lambda b, i: (b, i, 0)),
            pl.BlockSpec((1, ROWS_C * KNN, DIM), lambda b, i: (b, i, 0)),
            pl.BlockSpec((1, ROWS_C * KNN, DIM), lambda b, i: (b, i, 0)),
            _full((1, DIM)),
            _full((DIM, DIM)), _full((1, DIM)),
            _full((DIM, DIM)), _full((1, DIM)),
            _full((DIM, DIM)), _full((1, DIM)),
            _full((DIM, PDIM)), _full((1, PDIM)),
        ],
        out_specs=pl.BlockSpec((1, ROWS_C, PDIM), lambda b, i: (b, i, 0)),
        out_shape=jax.ShapeDtypeStruct((BATCH, NPTS, PDIM), f32),
    )(et, in_f, pdt,
      sg.reshape(BATCH, NPTS * KNN, DIM),
      ag.reshape(BATCH, NPTS * KNN, DIM),
      eg.reshape(BATCH, NPTS * KNN, DIM),
      row(del1_b), del2_w.T, row(del2_b),
      gam1_w.T, row(gam1_b), gam2_w.T, row(gam2_b),
      fc2_w.T, row(fc2_b))
    return out


# speedup vs baseline: 14.8156x; 1.3130x over previous
import functools

import jax
import jax.numpy as jnp
from jax import lax
from jax.experimental import pallas as pl
from jax.experimental.pallas import tpu as pltpu
from jax.experimental.pallas import tpu_sc as plsc

BATCH = 2
NPTS = 2048
KNN = 16
DIM = 256
PDIM = 64
XPAD = 16

ROWS_A = 256
ROWS_C = 64

SC_CORES = 2
SC_SUBCORES = 16
SC_WORKERS = SC_CORES * SC_SUBCORES
PAIRS = NPTS * KNN
PAIRS_PER_WORKER = PAIRS // SC_WORKERS
GATHER_CHUNK = 128
N_CHUNKS = PAIRS_PER_WORKER // GATHER_CHUNK


def _stage1_body(x_ref, xt_ref, inf_ref,
                 fc1wT, fc1b, phiwT, psiwT, alphawT,
                 dpt1wT, dpt1b, dpt2wT, dpt2b, del1pT,
                 idx_out, pd_out, s_out, a_out, e_out):
    xb = x_ref[0]
    xall_t = xt_ref[0]
    d = -2.0 * jnp.dot(xb, xall_t, preferred_element_type=jnp.float32)
    d = d + jnp.sum(xb * xb, axis=1, keepdims=True)
    d = d + jnp.sum(xall_t * xall_t, axis=0, keepdims=True)

    fiota = lax.broadcasted_iota(jnp.int32, d.shape, 1).astype(jnp.float32)
    cols = []
    for _ in range(KNN):
        m = jnp.min(d, axis=1, keepdims=True)
        ii = jnp.min(jnp.where(d == m, fiota, jnp.float32(NPTS)), axis=1)
        cols.append(ii)
        d = jnp.where(fiota == ii[:, None], jnp.float32(jnp.inf), d)
    idx = jnp.stack(cols, axis=1)
    idx_out[...] = idx.astype(jnp.int32)

    e_out[...] = jnp.dot(xb, del1pT[...], preferred_element_type=jnp.float32)
    f = jnp.dot(inf_ref[0], fc1wT[...], preferred_element_type=jnp.float32) + fc1b[...]
    dr = jnp.maximum(jnp.dot(f, dpt1wT[...], preferred_element_type=jnp.float32) + dpt1b[...], 0.0)
    dr = jnp.dot(dr, dpt2wT[...], preferred_element_type=jnp.float32) + dpt2b[...]
    pd_out[...] = jnp.dot(f, phiwT[...], preferred_element_type=jnp.float32) + dr
    s_out[...] = jnp.dot(f, psiwT[...], preferred_element_type=jnp.float32)
    a_out[...] = jnp.dot(f, alphawT[...], preferred_element_type=jnp.float32)


def _sc_gather_body(idx_hbm, s_hbm, a_hbm, e_hbm,
                    sg_out, ag_out, eg_out,
                    idx_v, sbuf, abuf, ebuf, sem_s, sem_a, sem_e):
    wid = lax.axis_index("s") * SC_CORES + lax.axis_index("c")
    base = wid * PAIRS_PER_WORKER
    pltpu.sync_copy(idx_hbm.at[pl.ds(base, PAIRS_PER_WORKER)], idx_v)

    def chunk(c, carry):
        off = c * GATHER_CHUNK
        isl = idx_v.at[pl.ds(off, GATHER_CHUNK)]
        cp_s = pltpu.async_copy(s_hbm.at[isl], sbuf, sem_s)
        cp_a = pltpu.async_copy(a_hbm.at[isl], abuf, sem_a)
        cp_e = pltpu.async_copy(e_hbm.at[isl], ebuf, sem_e)
        cp_s.wait()
        pltpu.sync_copy(sbuf, sg_out.at[pl.ds(base + off, GATHER_CHUNK)])
        cp_a.wait()
        pltpu.sync_copy(abuf, ag_out.at[pl.ds(base + off, GATHER_CHUNK)])
        cp_e.wait()
        pltpu.sync_copy(ebuf, eg_out.at[pl.ds(base + off, GATHER_CHUNK)])
        return carry

    lax.fori_loop(0, N_CHUNKS, chunk, 0)


def _stage3_body(e_ref, inf_ref, pd_ref, sg_ref, ag_ref, eg_ref,
                 del1b, del2wT, del2b,
                 gam1wT, gam1b, gam2wT, gam2b,
                 fc2wT, fc2b, out_ref):
    nb = ROWS_C
    en = e_ref[...]
    eg = eg_ref[...].reshape(nb, KNN, DIM)
    h = (en[:, None, :] - eg).reshape(nb * KNN, DIM) + del1b[...]
    u = jnp.maximum(h, 0.0)
    delta = jnp.dot(u, del2wT[...], preferred_element_type=jnp.float32) + del2b[...]

    pd = pd_ref[...]
    t = (pd[:, None, :] - sg_ref[...].reshape(nb, KNN, DIM)).reshape(nb * KNN, DIM) + delta
    g1 = jnp.maximum(jnp.dot(t, gam1wT[...], preferred_element_type=jnp.float32) + gam1b[...], 0.0)
    gamma = jnp.dot(g1, gam2wT[...], preferred_element_type=jnp.float32) + gam2b[...]

    s = gamma.reshape(nb, KNN, DIM) * (1.0 / (KNN ** 0.5))
    smax = jnp.max(s, axis=1, keepdims=True)
    e = jnp.exp(s - smax)
    rho = e / jnp.sum(e, axis=1, keepdims=True)

    ad = ag_ref[...].reshape(nb, KNN, DIM) + delta.reshape(nb, KNN, DIM)
    y = jnp.sum(rho * ad, axis=1)
    out_ref[...] = jnp.dot(y, fc2wT[...], preferred_element_type=jnp.float32) + fc2b[...] + inf_ref[0]


def _full(shape):
    nd = len(shape)
    return pl.BlockSpec(shape, lambda i: (0,) * nd)


def kernel(x, in_f, fc1_w, fc1_b, fc2_w, fc2_b, phi_w, psi_w, alpha_w,
           dpt1_w, dpt1_b, dpt2_w, dpt2_b, gam1_w, gam1_b, gam2_w, gam2_b,
           del1_w, del1_b, del2_w, del2_b):
    f32 = jnp.float32
    xpad = jnp.pad(x, ((0, 0), (0, 0), (0, XPAD - 3)))
    xt = jnp.transpose(xpad, (0, 2, 1))
    del1p = jnp.pad(del1_w, ((0, 0), (0, XPAD - 3)))

    row = lambda v: v.reshape(1, -1)

    def stage1(b):
        return pl.pallas_call(
            _stage1_body,
            grid=(NPTS // ROWS_A,),
            in_specs=[
                pl.BlockSpec((1, ROWS_A, XPAD), lambda i: (b, i, 0)),
                pl.BlockSpec((1, XPAD, NPTS), lambda i: (b, 0, 0)),
                pl.BlockSpec((1, ROWS_A, PDIM), lambda i: (b, i, 0)),
                _full((PDIM, DIM)), _full((1, DIM)),
                _full((DIM, DIM)), _full((DIM, DIM)), _full((DIM, DIM)),
                _full((DIM, DIM)), _full((1, DIM)),
                _full((DIM, DIM)), _full((1, DIM)),
                _full((XPAD, DIM)),
            ],
            out_specs=[
                pl.BlockSpec((ROWS_A, KNN), lambda i: (i, 0)),
                pl.BlockSpec((ROWS_A, DIM), lambda i: (i, 0)),
                pl.BlockSpec((ROWS_A, DIM), lambda i: (i, 0)),
                pl.BlockSpec((ROWS_A, DIM), lambda i: (i, 0)),
                pl.BlockSpec((ROWS_A, DIM), lambda i: (i, 0)),
            ],
            out_shape=[
                jax.ShapeDtypeStruct((NPTS, KNN), jnp.int32),
                jax.ShapeDtypeStruct((NPTS, DIM), f32),
                jax.ShapeDtypeStruct((NPTS, DIM), f32),
                jax.ShapeDtypeStruct((NPTS, DIM), f32),
                jax.ShapeDtypeStruct((NPTS, DIM), f32),
            ],
        )(xpad, xt, in_f,
          fc1_w.T, row(fc1_b), phi_w.T, psi_w.T, alpha_w.T,
          dpt1_w.T, row(dpt1_b), dpt2_w.T, row(dpt2_b), del1p.T)

    sc_gather = functools.partial(
        pl.kernel,
        out_type=[
            jax.ShapeDtypeStruct((PAIRS, DIM), f32),
            jax.ShapeDtypeStruct((PAIRS, DIM), f32),
            jax.ShapeDtypeStruct((PAIRS, DIM), f32),
        ],
        mesh=plsc.VectorSubcoreMesh(core_axis_name="c", subcore_axis_name="s"),
        scratch_types=[
            pltpu.VMEM((PAIRS_PER_WORKER,), jnp.int32),
            pltpu.VMEM((GATHER_CHUNK, DIM), f32),
            pltpu.VMEM((GATHER_CHUNK, DIM), f32),
            pltpu.VMEM((GATHER_CHUNK, DIM), f32),
            pltpu.SemaphoreType.DMA,
            pltpu.SemaphoreType.DMA,
            pltpu.SemaphoreType.DMA,
        ],
    )(_sc_gather_body)

    def stage3(b, et, pdt, sg, ag, eg):
        return pl.pallas_call(
            _stage3_body,
            grid=(NPTS // ROWS_C,),
            in_specs=[
                pl.BlockSpec((ROWS_C, DIM), lambda i: (i, 0)),
                pl.BlockSpec((1, ROWS_C, PDIM), lambda i: (b, i, 0)),
                pl.BlockSpec((ROWS_C, DIM), lambda i: (i, 0)),
                pl.BlockSpec((ROWS_C * KNN, DIM), lambda i: (i, 0)),
                pl.BlockSpec((ROWS_C * KNN, DIM), lambda i: (i, 0)),
                pl.BlockSpec((ROWS_C * KNN, DIM), lambda i: (i, 0)),
                _full((1, DIM)),
                _full((DIM, DIM)), _full((1, DIM)),
                _full((DIM, DIM)), _full((1, DIM)),
                _full((DIM, DIM)), _full((1, DIM)),
                _full((DIM, PDIM)), _full((1, PDIM)),
            ],
            out_specs=pl.BlockSpec((ROWS_C, PDIM), lambda i: (i, 0)),
            out_shape=jax.ShapeDtypeStruct((NPTS, PDIM), f32),
        )(et, in_f, pdt, sg, ag, eg,
          row(del1_b), del2_w.T, row(del2_b),
          gam1_w.T, row(gam1_b), gam2_w.T, row(gam2_b),
          fc2_w.T, row(fc2_b))

    outs = []
    gathered = [None, None]
    tabs = [None, None]
    for b in range(BATCH):
        idx, pdt, st, at, et = stage1(b)
        tabs[b] = (et, pdt)
        gathered[b] = sc_gather(idx.reshape(PAIRS), st, at, et)
    for b in range(BATCH):
        et, pdt = tabs[b]
        sg, ag, eg = gathered[b]
        outs.append(stage3(b, et, pdt, sg, ag, eg))
    return jnp.stack(outs)


# speedup vs baseline: 16.1318x; 1.0888x over previous
import functools

import jax
import jax.numpy as jnp
from jax import lax
from jax.experimental import pallas as pl
from jax.experimental.pallas import tpu as pltpu
from jax.experimental.pallas import tpu_sc as plsc

BATCH = 2
NPTS = 2048
KNN = 16
DIM = 256
PDIM = 64
XPAD = 16
XWIDE = 128

ROWS_A = 256
ROWS_C = 64

SC_CORES = 2
SC_SUBCORES = 16
SC_WORKERS = SC_CORES * SC_SUBCORES
PAIRS = NPTS * KNN
PAIRS_PER_WORKER = PAIRS // SC_WORKERS
GATHER_CHUNK = 128
N_CHUNKS = PAIRS_PER_WORKER // GATHER_CHUNK


def _stage1_body(x_ref, xt_ref, inf_ref,
                 fc1wT, fc1b, phiwT, psiwT, alphawT,
                 dpt1wT, dpt1b, dpt2wT, dpt2b,
                 idx_out, pd_out, s_out, a_out):
    xb = x_ref[0]
    xall_t = xt_ref[0]
    d = -2.0 * jnp.dot(xb, xall_t, preferred_element_type=jnp.float32)
    d = d + jnp.sum(xb * xb, axis=1, keepdims=True)
    d = d + jnp.sum(xall_t * xall_t, axis=0, keepdims=True)

    fiota = lax.broadcasted_iota(jnp.int32, d.shape, 1).astype(jnp.float32)
    cols = []
    for _ in range(KNN):
        m = jnp.min(d, axis=1, keepdims=True)
        ii = jnp.min(jnp.where(d == m, fiota, jnp.float32(NPTS)), axis=1)
        cols.append(ii)
        d = jnp.where(fiota == ii[:, None], jnp.float32(jnp.inf), d)
    idx = jnp.stack(cols, axis=1)
    idx_out[...] = idx.astype(jnp.int32)

    f = jnp.dot(inf_ref[0], fc1wT[...], preferred_element_type=jnp.float32) + fc1b[...]
    dr = jnp.maximum(jnp.dot(f, dpt1wT[...], preferred_element_type=jnp.float32) + dpt1b[...], 0.0)
    dr = jnp.dot(dr, dpt2wT[...], preferred_element_type=jnp.float32) + dpt2b[...]
    pd_out[...] = jnp.dot(f, phiwT[...], preferred_element_type=jnp.float32) + dr
    s_out[...] = jnp.dot(f, psiwT[...], preferred_element_type=jnp.float32)
    a_out[...] = jnp.dot(f, alphawT[...], preferred_element_type=jnp.float32)


def _sc_gather_body(idx_hbm, s_hbm, a_hbm, x_hbm,
                    sg_out, ag_out, xg_out,
                    idx_v, sbuf, abuf, xbuf, sem_s, sem_a, sem_x):
    wid = lax.axis_index("s") * SC_CORES + lax.axis_index("c")
    base = wid * PAIRS_PER_WORKER
    pltpu.sync_copy(idx_hbm.at[pl.ds(base, PAIRS_PER_WORKER)], idx_v)

    def chunk(c, carry):
        off = c * GATHER_CHUNK
        isl = idx_v.at[pl.ds(off, GATHER_CHUNK)]
        cp_s = pltpu.async_copy(s_hbm.at[isl], sbuf, sem_s)
        cp_a = pltpu.async_copy(a_hbm.at[isl], abuf, sem_a)
        cp_x = pltpu.async_copy(x_hbm.at[isl], xbuf, sem_x)
        cp_s.wait()
        pltpu.sync_copy(sbuf, sg_out.at[pl.ds(base + off, GATHER_CHUNK)])
        cp_a.wait()
        pltpu.sync_copy(abuf, ag_out.at[pl.ds(base + off, GATHER_CHUNK)])
        cp_x.wait()
        pltpu.sync_copy(xbuf, xg_out.at[pl.ds(base + off, GATHER_CHUNK)])
        return carry

    lax.fori_loop(0, N_CHUNKS, chunk, 0)


def _stage3_body(xn_ref, inf_ref, pd_ref, sg_ref, ag_ref, xg_ref,
                 del1pT, del1b, del2wT, del2b,
                 gam1wT, gam1b, gam2wT, gam2b,
                 fc2wT, fc2b, out_ref):
    nb = ROWS_C
    xn = xn_ref[0]
    xg = xg_ref[...].reshape(nb, KNN, XWIDE)
    xdiff = (xn[:, None, :] - xg).reshape(nb * KNN, XWIDE)
    h = jnp.dot(xdiff, del1pT[...], preferred_element_type=jnp.float32) + del1b[...]
    u = jnp.maximum(h, 0.0)
    delta = jnp.dot(u, del2wT[...], preferred_element_type=jnp.float32) + del2b[...]

    pd = pd_ref[...]
    t = (pd[:, None, :] - sg_ref[...].reshape(nb, KNN, DIM)).reshape(nb * KNN, DIM) + delta
    g1 = jnp.maximum(jnp.dot(t, gam1wT[...], preferred_element_type=jnp.float32) + gam1b[...], 0.0)
    gamma = jnp.dot(g1, gam2wT[...], preferred_element_type=jnp.float32) + gam2b[...]

    s = gamma.reshape(nb, KNN, DIM) * (1.0 / (KNN ** 0.5))
    smax = jnp.max(s, axis=1, keepdims=True)
    e = jnp.exp(s - smax)
    rho = e / jnp.sum(e, axis=1, keepdims=True)

    ad = ag_ref[...].reshape(nb, KNN, DIM) + delta.reshape(nb, KNN, DIM)
    y = jnp.sum(rho * ad, axis=1)
    out_ref[...] = jnp.dot(y, fc2wT[...], preferred_element_type=jnp.float32) + fc2b[...] + inf_ref[0]


def _full(shape):
    nd = len(shape)
    return pl.BlockSpec(shape, lambda i: (0,) * nd)


def kernel(x, in_f, fc1_w, fc1_b, fc2_w, fc2_b, phi_w, psi_w, alpha_w,
           dpt1_w, dpt1_b, dpt2_w, dpt2_b, gam1_w, gam1_b, gam2_w, gam2_b,
           del1_w, del1_b, del2_w, del2_b):
    f32 = jnp.float32
    xpad = jnp.pad(x, ((0, 0), (0, 0), (0, XPAD - 3)))
    xt = jnp.transpose(xpad, (0, 2, 1))
    xw = jnp.pad(x, ((0, 0), (0, 0), (0, XWIDE - 3)))
    del1p = jnp.pad(del1_w, ((0, 0), (0, XWIDE - 3)))

    row = lambda v: v.reshape(1, -1)

    def stage1(b):
        return pl.pallas_call(
            _stage1_body,
            grid=(NPTS // ROWS_A,),
            in_specs=[
                pl.BlockSpec((1, ROWS_A, XPAD), lambda i: (b, i, 0)),
                pl.BlockSpec((1, XPAD, NPTS), lambda i: (b, 0, 0)),
                pl.BlockSpec((1, ROWS_A, PDIM), lambda i: (b, i, 0)),
                _full((PDIM, DIM)), _full((1, DIM)),
                _full((DIM, DIM)), _full((DIM, DIM)), _full((DIM, DIM)),
                _full((DIM, DIM)), _full((1, DIM)),
                _full((DIM, DIM)), _full((1, DIM)),
            ],
            out_specs=[
                pl.BlockSpec((ROWS_A, KNN), lambda i: (i, 0)),
                pl.BlockSpec((ROWS_A, DIM), lambda i: (i, 0)),
                pl.BlockSpec((ROWS_A, DIM), lambda i: (i, 0)),
                pl.BlockSpec((ROWS_A, DIM), lambda i: (i, 0)),
            ],
            out_shape=[
                jax.ShapeDtypeStruct((NPTS, KNN), jnp.int32),
                jax.ShapeDtypeStruct((NPTS, DIM), f32),
                jax.ShapeDtypeStruct((NPTS, DIM), f32),
                jax.ShapeDtypeStruct((NPTS, DIM), f32),
            ],
        )(xpad, xt, in_f,
          fc1_w.T, row(fc1_b), phi_w.T, psi_w.T, alpha_w.T,
          dpt1_w.T, row(dpt1_b), dpt2_w.T, row(dpt2_b))

    sc_gather = functools.partial(
        pl.kernel,
        out_type=[
            jax.ShapeDtypeStruct((PAIRS, DIM), f32),
            jax.ShapeDtypeStruct((PAIRS, DIM), f32),
            jax.ShapeDtypeStruct((PAIRS, XWIDE), f32),
        ],
        mesh=plsc.VectorSubcoreMesh(core_axis_name="c", subcore_axis_name="s"),
        scratch_types=[
            pltpu.VMEM((PAIRS_PER_WORKER,), jnp.int32),
            pltpu.VMEM((GATHER_CHUNK, DIM), f32),
            pltpu.VMEM((GATHER_CHUNK, DIM), f32),
            pltpu.VMEM((GATHER_CHUNK, XWIDE), f32),
            pltpu.SemaphoreType.DMA,
            pltpu.SemaphoreType.DMA,
            pltpu.SemaphoreType.DMA,
        ],
    )(_sc_gather_body)

    def stage3(b, pdt, sg, ag, xg):
        return pl.pallas_call(
            _stage3_body,
            grid=(NPTS // ROWS_C,),
            in_specs=[
                pl.BlockSpec((1, ROWS_C, XWIDE), lambda i: (b, i, 0)),
                pl.BlockSpec((1, ROWS_C, PDIM), lambda i: (b, i, 0)),
                pl.BlockSpec((ROWS_C, DIM), lambda i: (i, 0)),
                pl.BlockSpec((ROWS_C * KNN, DIM), lambda i: (i, 0)),
                pl.BlockSpec((ROWS_C * KNN, DIM), lambda i: (i, 0)),
                pl.BlockSpec((ROWS_C * KNN, XWIDE), lambda i: (i, 0)),
                _full((XWIDE, DIM)), _full((1, DIM)),
                _full((DIM, DIM)), _full((1, DIM)),
                _full((DIM, DIM)), _full((1, DIM)),
                _full((DIM, DIM)), _full((1, DIM)),
                _full((DIM, PDIM)), _full((1, PDIM)),
            ],
            out_specs=pl.BlockSpec((ROWS_C, PDIM), lambda i: (i, 0)),
            out_shape=jax.ShapeDtypeStruct((NPTS, PDIM), f32),
        )(xw, in_f, pdt, sg, ag, xg,
          del1p.T, row(del1_b), del2_w.T, row(del2_b),
          gam1_w.T, row(gam1_b), gam2_w.T, row(gam2_b),
          fc2_w.T, row(fc2_b))

    outs = []
    gathered = [None, None]
    tabs = [None, None]
    for b in range(BATCH):
        idx, pdt, st, at = stage1(b)
        tabs[b] = pdt
        gathered[b] = sc_gather(idx.reshape(PAIRS), st, at, xw[b])
    for b in range(BATCH):
        sg, ag, xg = gathered[b]
        outs.append(stage3(b, tabs[b], sg, ag, xg))
    return jnp.stack(outs)
